# baseline (speedup 1.0000x reference)
import functools

import jax
import jax.numpy as jnp
from jax.experimental import pallas as pl
from jax.experimental.pallas import tpu as pltpu

_B, _T, _D = 16, 1024, 32
_K = 8192
_TT = 128
_BT = _B * _T
_G = _BT // _TT
_NG = _K // 128

_PREC = jax.lax.Precision.DEFAULT


def _dot(a, b, dims):
    return jax.lax.dot_general(
        a, b, (dims, ((), ())),
        preferred_element_type=jnp.float32, precision=_PREC)


def _round(res, cbn):
    neg1 = jnp.full((_TT, 2), -1.0, jnp.float32)
    res2m = jnp.concatenate([res + res, neg1], axis=1)
    s = _dot(res2m, cbn, ((1,), (1,)))
    m = jnp.max(s, axis=1, keepdims=True)
    e = jnp.exp(s - m)
    d = jnp.sum(e, axis=1, keepdims=True)
    q = _dot(e, cbn[:, :_D], ((1,), (0,))) * (1.0 / d)
    return q, m + jnp.log(d)


def _body(x_ref, cbn_ref, c2_ref, pint_ref, recon_ref, loss_ref):
    x = x_ref[...]
    c2 = c2_ref[...]

    q0, l0 = _round(x, cbn_ref[0])
    r1 = x - q0
    q1, l1 = _round(r1, cbn_ref[1])
    recon = q0 + q1
    recon_ref[...] = recon
    dr = recon - x
    loss_ref[...] = (jnp.sum(r1 * r1) + jnp.sum(dr * dr)).reshape(1, 1, 1)

    l0h = l0.astype(jnp.bfloat16).astype(jnp.float32)
    l1h = l1.astype(jnp.bfloat16).astype(jnp.float32)
    one = jnp.ones((_TT, 1), jnp.float32)
    a = jnp.concatenate(
        [x, r1, one, one, l0h, l0 - l0h, l1h, l1 - l1h], axis=1)
    p = jnp.exp(_dot(a, c2, ((1,), (0,))))
    pint_ref[...] = p.reshape(_TT, 128, 128)


@jax.jit
def kernel(x, codebooks):
    x2 = x.reshape(_BT, _D)
    cb = codebooks
    nrm = jnp.sum(cb * cb, axis=-1, keepdims=True)
    nrm_h = nrm.astype(jnp.bfloat16).astype(jnp.float32)
    cbn = jnp.concatenate([cb, nrm_h, nrm - nrm_h], axis=-1)

    zg = jnp.zeros((_NG, 128, _D), dtype=cb.dtype)
    cb0g = cb[0].reshape(_NG, 128, _D)
    cb1g = cb[1].reshape(_NG, 128, _D)
    top = 2.0 * jnp.stack([cb0g, zg], axis=1).reshape(2 * _K, _D)
    bot = 2.0 * jnp.stack([zg, cb1g], axis=1).reshape(2 * _K, _D)
    n_int = jnp.stack(
        [nrm[0].reshape(_NG, 128), nrm[1].reshape(_NG, 128)],
        axis=1).reshape(2 * _K, 1)
    n_h = n_int.astype(jnp.bfloat16).astype(jnp.float32)
    rbit = (jnp.arange(2 * _K, dtype=jnp.int32)[:, None] >> 7) & 1
    mask_e = jnp.where(rbit == 0, -1.0, 0.0).astype(jnp.float32)
    mask_o = jnp.where(rbit == 0, 0.0, -1.0).astype(jnp.float32)
    c2 = jnp.concatenate(
        [top, bot, -n_h, n_h - n_int, mask_e, mask_e, mask_o, mask_o],
        axis=1).T

    pint, recon2, losses = pl.pallas_call(
        _body,
        grid=(_G,),
        in_specs=[
            pl.BlockSpec((_TT, _D), lambda i: (i, 0)),
            pl.BlockSpec((2, _K, _D + 2), lambda i: (0, 0, 0)),
            pl.BlockSpec((2 * _D + 6, 2 * _K), lambda i: (0, 0)),
        ],
        out_specs=[
            pl.BlockSpec((_TT, 128, 128), lambda i: (i, 0, 0)),
            pl.BlockSpec((_TT, _D), lambda i: (i, 0)),
            pl.BlockSpec((1, 1, 1), lambda i: (i, 0, 0)),
        ],
        out_shape=[
            jax.ShapeDtypeStruct((_BT, 128, 128), jnp.float32),
            jax.ShapeDtypeStruct((_BT, _D), jnp.float32),
            jax.ShapeDtypeStruct((_G, 1, 1), jnp.float32),
        ],
        compiler_params=pltpu.CompilerParams(
            dimension_semantics=("arbitrary",),
        ),
    )(x2, cbn, c2)
    index_probs = (pint.reshape(_B, _T, _NG, 2, 128)
                   .transpose(0, 1, 2, 4, 3)
                   .reshape(_B, _T, _K, 2))
    recon = recon2.reshape(_B, _T, _D)
    loss = jnp.sum(losses) * (1.25 / (_BT * _D))
    return recon, index_probs, loss
